# baseline (speedup 1.0000x reference)
import jax
import jax.numpy as jnp
import numpy as np
from jax.experimental import pallas as pl
from jax.experimental.pallas import tpu as pltpu

_B, _N, _D = 2048, 16, 64
_G = 64
_L = _G * _N
_GRID = _B // _G


def _lrelu(x):
    return jnp.maximum(x, 0.01 * x)


def _pad8(w):
    return jnp.pad(w, ((0, 8 - w.shape[0]), (0, 0)))


def _body(flat_ref, We1_ref, be1_ref, We2_ref, be2_ref, Wh1h_ref,
          Wh1t2_ref, Wh1s_ref, bh1_ref, whard_ref, bhard_ref, Wq_ref,
          Wkt_ref, Wks_ref, Wvt_ref, Wvs_ref, bv_ref, Ws1q_ref, Ws1k_ref,
          bs1_ref, ws2_ref, bs2_ref, Wd1h_ref, Wd1o_ref, bd1_ref, Wd2_ref,
          bd2_ref, out_ref):
    f32 = jnp.float32
    dot = lambda a, b: jnp.dot(a, b, preferred_element_type=f32)
    raw = flat_ref[0]
    raw = jnp.pad(raw, ((0, 0), (0, 3)))
    raw = raw.reshape(_G, _N, 8).swapaxes(0, 1).reshape(_L, 8)
    flat = raw.T

    h1 = _lrelu(dot(We1_ref[...], flat) + be1_ref[...])
    h = _lrelu(dot(We2_ref[...], h1) + be2_ref[...])

    tpart = dot(Wh1h_ref[...], h) - dot(Wh1t2_ref[...], flat) + bh1_ref[...]
    kt = dot(Wkt_ref[...], flat)
    vt = dot(Wvt_ref[...], flat) + bv_ref[...]
    qs1 = dot(Ws1q_ref[...], dot(Wq_ref[...], h)) + bs1_ref[...]

    spart = dot(Wh1s_ref[...], flat)
    ks = dot(Wks_ref[...], flat)
    vs = dot(Wvs_ref[...], flat)

    whard = whard_ref[...]
    ws2 = ws2_ref[...]
    lane_t = jax.lax.broadcasted_iota(jnp.int32, (1, _L), 1) // _G

    hard_rows, score_rows, v4s = [], [], []
    for s in range(_N):
        st = slice(s * _G, (s + 1) * _G)
        hh = jnp.maximum(tpart + pltpu.repeat(spart[:, st], _N, 1), 0.0)
        hard_rows.append(jnp.sum(hh * whard, axis=0, keepdims=True))
        k4 = _lrelu(kt + pltpu.repeat(ks[:, st], _N, 1))
        spre = jnp.maximum(dot(Ws1k_ref[...], k4) + qs1, 0.0)
        sc = jnp.sum(spre * ws2, axis=0, keepdims=True) + bs2_ref[0, 0]
        score_rows.append(jnp.where(lane_t == s, -1e30, sc))
        v4s.append(_lrelu(vt + pltpu.repeat(vs[:, st], _N, 1)))

    scores = jnp.concatenate(score_rows, axis=0)
    m = jnp.max(scores, axis=0, keepdims=True)
    ex = jnp.exp(scores - m)
    hard_logit = jnp.concatenate(hard_rows, axis=0) + bhard_ref[0, 0]
    w = (ex / jnp.sum(ex, axis=0, keepdims=True)) * jax.nn.sigmoid(hard_logit)

    agg = w[0:1] * v4s[0]
    for s in range(1, _N):
        agg = agg + w[s:s + 1] * v4s[s]

    dec = _lrelu(dot(Wd1h_ref[...], h) + dot(Wd1o_ref[...], agg)
                 + bd1_ref[...])
    dec = _lrelu(dot(Wd2_ref[...], dec) + bd2_ref[...])
    out_ref[0] = dec.T.reshape(_N, _G, 2 * _D).swapaxes(0, 1)


def kernel(embedding, W_e1, b_e1, W_e2, b_e2, W_h1, b_h1, W_h2, b_h2,
           W_he, b_he, W_q, W_k, W_v, b_v, W_s1, b_s1, W_s2, b_s2,
           W_d1, b_d1, W_d2, b_d2):
    flat_t = embedding.reshape(_GRID, _L, 5)

    whe_diff = W_he[:, 1] - W_he[:, 0]
    col1 = lambda b: b.reshape(-1, 1)
    weights = (
        _pad8(W_e1).T, col1(b_e1), W_e2.T, col1(b_e2),
        W_h1[:_D].T, _pad8(W_h1[_D + 5:]).T,
        (_pad8(W_h1[_D:_D + 5]) + _pad8(W_h1[_D + 5:])).T, col1(b_h1),
        (W_h2 @ whe_diff).reshape(_D, 1),
        (b_h2 @ whe_diff + b_he[1] - b_he[0]).reshape(1, 1),
        W_q.T,
        _pad8(W_k[:5]).T, _pad8(W_k[5:]).T,
        _pad8(W_v[:5]).T, _pad8(W_v[5:]).T, col1(b_v),
        W_s1[:_D].T, W_s1[_D:].T, col1(b_s1),
        W_s2.reshape(_D, 1), b_s2.reshape(1, 1),
        W_d1[:_D].T, W_d1[_D:].T, col1(b_d1),
        W_d2.T, col1(b_d2),
    )
    in_specs = [pl.BlockSpec((1, _L, 5), lambda i: (i, 0, 0))] + [
        pl.BlockSpec(w.shape, lambda i: (0, 0)) for w in weights
    ]
    out = pl.pallas_call(
        _body,
        grid=(_GRID,),
        in_specs=in_specs,
        out_specs=pl.BlockSpec((1, _G, _N, 2 * _D), lambda i: (i, 0, 0, 0)),
        out_shape=jax.ShapeDtypeStruct((_GRID, _G, _N, 2 * _D), jnp.float32),
        compiler_params=pltpu.CompilerParams(
            dimension_semantics=("parallel",)),
    )(flat_t, *weights)
    return out.reshape(_B * _N, 2 * _D)


# speedup vs baseline: 1.1917x; 1.1917x over previous
import jax
import jax.numpy as jnp
import numpy as np
from jax.experimental import pallas as pl
from jax.experimental.pallas import tpu as pltpu

_B, _N, _D = 2048, 16, 64
_G = 128
_L = _G * _N
_GRID = _B // _G


def _lrelu(x):
    return jnp.maximum(x, 0.01 * x)


def _pad8(w):
    return jnp.pad(w, ((0, 8 - w.shape[0]), (0, 0)))


def _body(flat_ref, We1_ref, be1_ref, We2_ref, be2_ref, Wh1h_ref,
          Wh1t2_ref, Wh1s_ref, bh1_ref, whard_ref, bhard_ref, Wq_ref,
          Wkt_ref, Wks_ref, Wvt_ref, Wvs_ref, bv_ref, Ws1q_ref, Ws1k_ref,
          bs1_ref, ws2_ref, bs2_ref, Wd1h_ref, Wd1o_ref, bd1_ref, Wd2_ref,
          bd2_ref, out_ref):
    f32 = jnp.float32
    dot = lambda a, b: jnp.dot(a, b, preferred_element_type=f32)
    raw = flat_ref[0]
    raw = jnp.pad(raw, ((0, 0), (0, 3)))
    raw = raw.reshape(_G, _N, 8).swapaxes(0, 1).reshape(_L, 8)
    flat = raw.T

    h1 = _lrelu(dot(We1_ref[...], flat) + be1_ref[...])
    h = _lrelu(dot(We2_ref[...], h1) + be2_ref[...])

    tpart = dot(Wh1h_ref[...], h) - dot(Wh1t2_ref[...], flat) + bh1_ref[...]
    kt = dot(Wkt_ref[...], flat)
    vt = dot(Wvt_ref[...], flat) + bv_ref[...]
    qs1 = dot(Ws1q_ref[...], dot(Wq_ref[...], h)) + bs1_ref[...]

    spart = dot(Wh1s_ref[...], flat)
    ks = dot(Wks_ref[...], flat)
    vs = dot(Wvs_ref[...], flat)

    whard = whard_ref[...]
    ws2 = ws2_ref[...]
    lane_t = jax.lax.broadcasted_iota(jnp.int32, (1, _L), 1) // _G

    hard_rows, score_rows, v4s = [], [], []
    for s in range(_N):
        st = slice(s * _G, (s + 1) * _G)
        hh = jnp.maximum(tpart + pltpu.repeat(spart[:, st], _N, 1), 0.0)
        hard_rows.append(jnp.sum(hh * whard, axis=0, keepdims=True))
        k4 = _lrelu(kt + pltpu.repeat(ks[:, st], _N, 1))
        spre = jnp.maximum(dot(Ws1k_ref[...], k4) + qs1, 0.0)
        sc = jnp.sum(spre * ws2, axis=0, keepdims=True) + bs2_ref[0, 0]
        score_rows.append(jnp.where(lane_t == s, -1e30, sc))
        v4s.append(_lrelu(vt + pltpu.repeat(vs[:, st], _N, 1)))

    scores = jnp.concatenate(score_rows, axis=0)
    m = jnp.max(scores, axis=0, keepdims=True)
    ex = jnp.exp(scores - m)
    hard_logit = jnp.concatenate(hard_rows, axis=0) + bhard_ref[0, 0]
    w = (ex / jnp.sum(ex, axis=0, keepdims=True)) * jax.nn.sigmoid(hard_logit)

    agg = w[0:1] * v4s[0]
    for s in range(1, _N):
        agg = agg + w[s:s + 1] * v4s[s]

    dec = _lrelu(dot(Wd1h_ref[...], h) + dot(Wd1o_ref[...], agg)
                 + bd1_ref[...])
    dec = _lrelu(dot(Wd2_ref[...], dec) + bd2_ref[...])
    out_ref[0] = dec.T.reshape(_N, _G, 2 * _D).swapaxes(0, 1)


def kernel(embedding, W_e1, b_e1, W_e2, b_e2, W_h1, b_h1, W_h2, b_h2,
           W_he, b_he, W_q, W_k, W_v, b_v, W_s1, b_s1, W_s2, b_s2,
           W_d1, b_d1, W_d2, b_d2):
    flat_t = embedding.reshape(_GRID, _L, 5)

    whe_diff = W_he[:, 1] - W_he[:, 0]
    col1 = lambda b: b.reshape(-1, 1)
    weights = (
        _pad8(W_e1).T, col1(b_e1), W_e2.T, col1(b_e2),
        W_h1[:_D].T, _pad8(W_h1[_D + 5:]).T,
        (_pad8(W_h1[_D:_D + 5]) + _pad8(W_h1[_D + 5:])).T, col1(b_h1),
        (W_h2 @ whe_diff).reshape(_D, 1),
        (b_h2 @ whe_diff + b_he[1] - b_he[0]).reshape(1, 1),
        W_q.T,
        _pad8(W_k[:5]).T, _pad8(W_k[5:]).T,
        _pad8(W_v[:5]).T, _pad8(W_v[5:]).T, col1(b_v),
        W_s1[:_D].T, W_s1[_D:].T, col1(b_s1),
        W_s2.reshape(_D, 1), b_s2.reshape(1, 1),
        W_d1[:_D].T, W_d1[_D:].T, col1(b_d1),
        W_d2.T, col1(b_d2),
    )
    in_specs = [pl.BlockSpec((1, _L, 5), lambda i: (i, 0, 0))] + [
        pl.BlockSpec(w.shape, lambda i: (0, 0)) for w in weights
    ]
    out = pl.pallas_call(
        _body,
        grid=(_GRID,),
        in_specs=in_specs,
        out_specs=pl.BlockSpec((1, _G, _N, 2 * _D), lambda i: (i, 0, 0, 0)),
        out_shape=jax.ShapeDtypeStruct((_GRID, _G, _N, 2 * _D), jnp.float32),
        compiler_params=pltpu.CompilerParams(
            dimension_semantics=("parallel",)),
    )(flat_t, *weights)
    return out.reshape(_B * _N, 2 * _D)


# speedup vs baseline: 1.1965x; 1.0040x over previous
import jax
import jax.numpy as jnp
import numpy as np
from jax.experimental import pallas as pl
from jax.experimental.pallas import tpu as pltpu

_B, _N, _D = 2048, 16, 64
_G = 256
_L = _G * _N
_GRID = _B // _G


def _lrelu(x):
    return jnp.maximum(x, 0.01 * x)


def _pad8(w):
    return jnp.pad(w, ((0, 8 - w.shape[0]), (0, 0)))


def _body(flat_ref, We1_ref, be1_ref, We2_ref, be2_ref, Wh1h_ref,
          Wh1t2_ref, Wh1s_ref, bh1_ref, whard_ref, bhard_ref, Wq_ref,
          Wkt_ref, Wks_ref, Wvt_ref, Wvs_ref, bv_ref, Ws1q_ref, Ws1k_ref,
          bs1_ref, ws2_ref, bs2_ref, Wd1h_ref, Wd1o_ref, bd1_ref, Wd2_ref,
          bd2_ref, out_ref):
    f32 = jnp.float32
    dot = lambda a, b: jnp.dot(a, b, preferred_element_type=f32)
    raw = flat_ref[0]
    raw = jnp.pad(raw, ((0, 0), (0, 3)))
    raw = raw.reshape(_G, _N, 8).swapaxes(0, 1).reshape(_L, 8)
    flat = raw.T

    h1 = _lrelu(dot(We1_ref[...], flat) + be1_ref[...])
    h = _lrelu(dot(We2_ref[...], h1) + be2_ref[...])

    tpart = dot(Wh1h_ref[...], h) - dot(Wh1t2_ref[...], flat) + bh1_ref[...]
    kt = dot(Wkt_ref[...], flat)
    vt = dot(Wvt_ref[...], flat) + bv_ref[...]
    qs1 = dot(Ws1q_ref[...], dot(Wq_ref[...], h)) + bs1_ref[...]

    spart = dot(Wh1s_ref[...], flat)
    ks = dot(Wks_ref[...], flat)
    vs = dot(Wvs_ref[...], flat)

    whard = whard_ref[...]
    ws2 = ws2_ref[...]
    lane_t = jax.lax.broadcasted_iota(jnp.int32, (1, _L), 1) // _G

    hard_rows, score_rows, v4s = [], [], []
    for s in range(_N):
        st = slice(s * _G, (s + 1) * _G)
        hh = jnp.maximum(tpart + pltpu.repeat(spart[:, st], _N, 1), 0.0)
        hard_rows.append(jnp.sum(hh * whard, axis=0, keepdims=True))
        k4 = _lrelu(kt + pltpu.repeat(ks[:, st], _N, 1))
        spre = jnp.maximum(dot(Ws1k_ref[...], k4) + qs1, 0.0)
        sc = jnp.sum(spre * ws2, axis=0, keepdims=True) + bs2_ref[0, 0]
        score_rows.append(jnp.where(lane_t == s, -1e30, sc))
        v4s.append(_lrelu(vt + pltpu.repeat(vs[:, st], _N, 1)))

    scores = jnp.concatenate(score_rows, axis=0)
    m = jnp.max(scores, axis=0, keepdims=True)
    ex = jnp.exp(scores - m)
    hard_logit = jnp.concatenate(hard_rows, axis=0) + bhard_ref[0, 0]
    w = (ex / jnp.sum(ex, axis=0, keepdims=True)) * jax.nn.sigmoid(hard_logit)

    agg = w[0:1] * v4s[0]
    for s in range(1, _N):
        agg = agg + w[s:s + 1] * v4s[s]

    dec = _lrelu(dot(Wd1h_ref[...], h) + dot(Wd1o_ref[...], agg)
                 + bd1_ref[...])
    dec = _lrelu(dot(Wd2_ref[...], dec) + bd2_ref[...])
    out_ref[0] = dec.T.reshape(_N, _G, 2 * _D).swapaxes(0, 1)


def kernel(embedding, W_e1, b_e1, W_e2, b_e2, W_h1, b_h1, W_h2, b_h2,
           W_he, b_he, W_q, W_k, W_v, b_v, W_s1, b_s1, W_s2, b_s2,
           W_d1, b_d1, W_d2, b_d2):
    flat_t = embedding.reshape(_GRID, _L, 5)

    whe_diff = W_he[:, 1] - W_he[:, 0]
    col1 = lambda b: b.reshape(-1, 1)
    weights = (
        _pad8(W_e1).T, col1(b_e1), W_e2.T, col1(b_e2),
        W_h1[:_D].T, _pad8(W_h1[_D + 5:]).T,
        (_pad8(W_h1[_D:_D + 5]) + _pad8(W_h1[_D + 5:])).T, col1(b_h1),
        (W_h2 @ whe_diff).reshape(_D, 1),
        (b_h2 @ whe_diff + b_he[1] - b_he[0]).reshape(1, 1),
        W_q.T,
        _pad8(W_k[:5]).T, _pad8(W_k[5:]).T,
        _pad8(W_v[:5]).T, _pad8(W_v[5:]).T, col1(b_v),
        W_s1[:_D].T, W_s1[_D:].T, col1(b_s1),
        W_s2.reshape(_D, 1), b_s2.reshape(1, 1),
        W_d1[:_D].T, W_d1[_D:].T, col1(b_d1),
        W_d2.T, col1(b_d2),
    )
    in_specs = [pl.BlockSpec((1, _L, 5), lambda i: (i, 0, 0))] + [
        pl.BlockSpec(w.shape, lambda i: (0, 0)) for w in weights
    ]
    out = pl.pallas_call(
        _body,
        grid=(_GRID,),
        in_specs=in_specs,
        out_specs=pl.BlockSpec((1, _G, _N, 2 * _D), lambda i: (i, 0, 0, 0)),
        out_shape=jax.ShapeDtypeStruct((_GRID, _G, _N, 2 * _D), jnp.float32),
        compiler_params=pltpu.CompilerParams(
            dimension_semantics=("parallel",)),
    )(flat_t, *weights)
    return out.reshape(_B * _N, 2 * _D)


# speedup vs baseline: 1.2026x; 1.0051x over previous
import jax
import jax.numpy as jnp
import numpy as np
from jax.experimental import pallas as pl
from jax.experimental.pallas import tpu as pltpu

_B, _N, _D = 2048, 16, 64
_G = 256
_L = _G * _N
_GRID = _B // _G


def _lrelu(x):
    return jnp.maximum(x, 0.01 * x)


def _pad8(w):
    return jnp.pad(w, ((0, 8 - w.shape[0]), (0, 0)))


def _body(flat_ref, We1_ref, be1_ref, We2_ref, be2_ref, Wh1h_ref,
          Wh1t2_ref, Wh1s_ref, bh1_ref, whard_ref, bhard_ref, Wq_ref,
          Wkt_ref, Wks_ref, Wvt_ref, Wvs_ref, bv_ref, Ws1q_ref, Ws1k_ref,
          bs1_ref, ws2_ref, bs2_ref, Wd1h_ref, Wd1o_ref, bd1_ref, Wd2_ref,
          bd2_ref, out_ref):
    f32 = jnp.float32
    dot = lambda a, b: jnp.dot(a, b, preferred_element_type=f32)
    dotT = lambda w, x: jax.lax.dot_general(
        w, x, (((0,), (0,)), ((), ())), preferred_element_type=f32)
    raw = flat_ref[0]
    raw = jnp.pad(raw, ((0, 0), (0, 3)))
    raw = raw.reshape(_G, _N, 8).swapaxes(0, 1).reshape(_L, 8)
    flat = raw.T

    h1 = _lrelu(dotT(We1_ref[...], flat) + be1_ref[...])
    h = _lrelu(dotT(We2_ref[...], h1) + be2_ref[...])

    tpart = (dotT(Wh1h_ref[...], h) - dotT(Wh1t2_ref[...], flat)
             + bh1_ref[...])
    kt = dotT(Wkt_ref[...], flat)
    vt = dotT(Wvt_ref[...], flat) + bv_ref[...]
    qs1 = dotT(Ws1q_ref[...], dotT(Wq_ref[...], h)) + bs1_ref[...]

    spart = dotT(Wh1s_ref[...], flat)
    ks = dotT(Wks_ref[...], flat)
    vs = dotT(Wvs_ref[...], flat)

    whard = whard_ref[...]
    ws2 = ws2_ref[...]
    lane_t = jax.lax.broadcasted_iota(jnp.int32, (1, _L), 1) // _G

    hard_rows, score_rows, v4s = [], [], []
    for s in range(_N):
        st = slice(s * _G, (s + 1) * _G)
        hh = jnp.maximum(tpart + pltpu.repeat(spart[:, st], _N, 1), 0.0)
        hard_rows.append(jnp.sum(hh * whard, axis=0, keepdims=True))
        k4 = _lrelu(kt + pltpu.repeat(ks[:, st], _N, 1))
        spre = jnp.maximum(dotT(Ws1k_ref[...], k4) + qs1, 0.0)
        sc = jnp.sum(spre * ws2, axis=0, keepdims=True) + bs2_ref[0, 0]
        score_rows.append(jnp.where(lane_t == s, -1e30, sc))
        v4s.append(_lrelu(vt + pltpu.repeat(vs[:, st], _N, 1)))

    scores = jnp.concatenate(score_rows, axis=0)
    m = jnp.max(scores, axis=0, keepdims=True)
    ex = jnp.exp(scores - m)
    hard_logit = jnp.concatenate(hard_rows, axis=0) + bhard_ref[0, 0]
    w = (ex / jnp.sum(ex, axis=0, keepdims=True)) * jax.nn.sigmoid(hard_logit)

    agg = w[0:1] * v4s[0]
    for s in range(1, _N):
        agg = agg + w[s:s + 1] * v4s[s]

    dec = _lrelu(dotT(Wd1h_ref[...], h) + dotT(Wd1o_ref[...], agg)
                 + bd1_ref[...])
    dec = _lrelu(dotT(Wd2_ref[...], dec) + bd2_ref[...])
    out_ref[0] = dec.T.reshape(_N, _G, 2 * _D).swapaxes(0, 1)


def kernel(embedding, W_e1, b_e1, W_e2, b_e2, W_h1, b_h1, W_h2, b_h2,
           W_he, b_he, W_q, W_k, W_v, b_v, W_s1, b_s1, W_s2, b_s2,
           W_d1, b_d1, W_d2, b_d2):
    flat_t = embedding.reshape(_GRID, _L, 5)

    whe_diff = W_he[:, 1] - W_he[:, 0]
    col1 = lambda b: b.reshape(-1, 1)
    weights = (
        _pad8(W_e1), col1(b_e1), W_e2, col1(b_e2),
        W_h1[:_D], _pad8(W_h1[_D + 5:]),
        _pad8(W_h1[_D:_D + 5]) + _pad8(W_h1[_D + 5:]), col1(b_h1),
        (W_h2 @ whe_diff).reshape(_D, 1),
        (b_h2 @ whe_diff + b_he[1] - b_he[0]).reshape(1, 1),
        W_q,
        _pad8(W_k[:5]), _pad8(W_k[5:]),
        _pad8(W_v[:5]), _pad8(W_v[5:]), col1(b_v),
        W_s1[:_D], W_s1[_D:], col1(b_s1),
        W_s2.reshape(_D, 1), b_s2.reshape(1, 1),
        W_d1[:_D], W_d1[_D:], col1(b_d1),
        W_d2, col1(b_d2),
    )
    in_specs = [pl.BlockSpec((1, _L, 5), lambda i: (i, 0, 0))] + [
        pl.BlockSpec(w.shape, lambda i: (0, 0)) for w in weights
    ]
    out = pl.pallas_call(
        _body,
        grid=(_GRID,),
        in_specs=in_specs,
        out_specs=pl.BlockSpec((1, _G, _N, 2 * _D), lambda i: (i, 0, 0, 0)),
        out_shape=jax.ShapeDtypeStruct((_GRID, _G, _N, 2 * _D), jnp.float32),
        compiler_params=pltpu.CompilerParams(
            dimension_semantics=("parallel",)),
    )(flat_t, *weights)
    return out.reshape(_B * _N, 2 * _D)


# speedup vs baseline: 1.2213x; 1.0156x over previous
import jax
import jax.numpy as jnp
import numpy as np
from jax.experimental import pallas as pl
from jax.experimental.pallas import tpu as pltpu

_B, _N, _D = 2048, 16, 64
_G = 256
_L = _G * _N
_GRID = _B // _G


def _lrelu(x):
    return jnp.maximum(x, 0.01 * x)


def _pad8(w):
    return jnp.pad(w, ((0, 8 - w.shape[0]), (0, 0)))


def _body(flat_ref, We1_ref, be1_ref, We2_ref, be2_ref, Wh1h_ref,
          Wh1t2_ref, Wh1s_ref, bh1_ref, whard_ref, bhard_ref, Wq_ref,
          Wkt_ref, Wks_ref, Wvt_ref, Wvs_ref, bv_ref, Ws1q_ref, Ws1k_ref,
          bs1_ref, ws2_ref, bs2_ref, Wd1h_ref, Wd1o_ref, bd1_ref, Wd2_ref,
          bd2_ref, out_ref):
    f32 = jnp.float32
    dot = lambda a, b: jnp.dot(a, b, preferred_element_type=f32)
    dotT = lambda w, x: jax.lax.dot_general(
        w, x, (((0,), (0,)), ((), ())), preferred_element_type=f32)
    dotRT = lambda w, x: jax.lax.dot_general(
        w, x, (((0,), (1,)), ((), ())), preferred_element_type=f32)
    raw = flat_ref[0]
    raw = raw.reshape(_G, _N, 5).swapaxes(0, 1).reshape(_L, 5)

    h1 = _lrelu(dotRT(We1_ref[...], raw) + be1_ref[...])
    h = _lrelu(dotT(We2_ref[...], h1) + be2_ref[...])

    raw7 = jnp.concatenate([raw, raw[:, :2]], axis=1)
    tpart = (dotT(Wh1h_ref[...], h) - dotRT(Wh1t2_ref[...], raw[:, :2])
             + bh1_ref[...])
    kt = dotRT(Wkt_ref[...], raw)
    vt = dotRT(Wvt_ref[...], raw) + bv_ref[...]
    qs1 = dotT(Ws1q_ref[...], dotT(Wq_ref[...], h)) + bs1_ref[...]

    spart = dotRT(Wh1s_ref[...], raw7)
    ks = dotRT(Wks_ref[...], raw)
    vs = dotRT(Wvs_ref[...], raw)

    whard = whard_ref[...]
    ws2 = ws2_ref[...]
    lane_t = jax.lax.broadcasted_iota(jnp.int32, (1, _L), 1) // _G

    hard_rows, score_rows, v4s = [], [], []
    for s in range(_N):
        st = slice(s * _G, (s + 1) * _G)
        hh = jnp.maximum(tpart + pltpu.repeat(spart[:, st], _N, 1), 0.0)
        hard_rows.append(jnp.sum(hh * whard, axis=0, keepdims=True))
        k4 = _lrelu(kt + pltpu.repeat(ks[:, st], _N, 1))
        spre = jnp.maximum(dotT(Ws1k_ref[...], k4) + qs1, 0.0)
        sc = jnp.sum(spre * ws2, axis=0, keepdims=True) + bs2_ref[0, 0]
        score_rows.append(jnp.where(lane_t == s, -1e30, sc))
        v4s.append(_lrelu(vt + pltpu.repeat(vs[:, st], _N, 1)))

    scores = jnp.concatenate(score_rows, axis=0)
    m = jnp.max(scores, axis=0, keepdims=True)
    ex = jnp.exp(scores - m)
    hard_logit = jnp.concatenate(hard_rows, axis=0) + bhard_ref[0, 0]
    w = (ex / jnp.sum(ex, axis=0, keepdims=True)) * jax.nn.sigmoid(hard_logit)

    agg = w[0:1] * v4s[0]
    for s in range(1, _N):
        agg = agg + w[s:s + 1] * v4s[s]

    dec = _lrelu(dotT(Wd1h_ref[...], h) + dotT(Wd1o_ref[...], agg)
                 + bd1_ref[...])
    dec = _lrelu(dotT(Wd2_ref[...], dec) + bd2_ref[...])
    out_ref[0] = dec.T.reshape(_N, _G, 2 * _D).swapaxes(0, 1)


def kernel(embedding, W_e1, b_e1, W_e2, b_e2, W_h1, b_h1, W_h2, b_h2,
           W_he, b_he, W_q, W_k, W_v, b_v, W_s1, b_s1, W_s2, b_s2,
           W_d1, b_d1, W_d2, b_d2):
    flat_t = embedding.reshape(_GRID, _L, 5)

    whe_diff = W_he[:, 1] - W_he[:, 0]
    col1 = lambda b: b.reshape(-1, 1)
    weights = (
        W_e1, col1(b_e1), W_e2, col1(b_e2),
        W_h1[:_D], W_h1[_D + 5:],
        W_h1[_D:], col1(b_h1),
        (W_h2 @ whe_diff).reshape(_D, 1),
        (b_h2 @ whe_diff + b_he[1] - b_he[0]).reshape(1, 1),
        W_q,
        W_k[:5], W_k[5:],
        W_v[:5], W_v[5:], col1(b_v),
        W_s1[:_D], W_s1[_D:], col1(b_s1),
        W_s2.reshape(_D, 1), b_s2.reshape(1, 1),
        W_d1[:_D], W_d1[_D:], col1(b_d1),
        W_d2, col1(b_d2),
    )
    in_specs = [pl.BlockSpec((1, _L, 5), lambda i: (i, 0, 0))] + [
        pl.BlockSpec(w.shape, lambda i: (0, 0)) for w in weights
    ]
    out = pl.pallas_call(
        _body,
        grid=(_GRID,),
        in_specs=in_specs,
        out_specs=pl.BlockSpec((1, _G, _N, 2 * _D), lambda i: (i, 0, 0, 0)),
        out_shape=jax.ShapeDtypeStruct((_GRID, _G, _N, 2 * _D), jnp.float32),
        compiler_params=pltpu.CompilerParams(
            dimension_semantics=("parallel",)),
    )(flat_t, *weights)
    return out.reshape(_B * _N, 2 * _D)


# speedup vs baseline: 1.2224x; 1.0009x over previous
import jax
import jax.numpy as jnp
from jax.experimental import pallas as pl
from jax.experimental.pallas import tpu as pltpu

_B, _N, _D = 2048, 16, 64
_G = 256
_L = _G * _N
_GRID = _B // _G


def _lrelu(x):
    return jnp.maximum(x, 0.01 * x)


def _body(flat_ref, We1_ref, be1_ref, We2_ref, be2_ref, Wh1h_ref,
          Wh1t2_ref, Wh1s_ref, bh1_ref, whard_ref, bhard_ref, Wq_ref,
          Wkt_ref, Wks_ref, Wvt_ref, Wvs_ref, bv_ref, Ws1q_ref, Ws1k_ref,
          bs1_ref, ws2_ref, bs2_ref, Wd1h_ref, Wd1o_ref, bd1_ref, Wd2_ref,
          bd2_ref, out_ref):
    f32 = jnp.float32
    dotT = lambda w, x: jax.lax.dot_general(
        w, x, (((0,), (0,)), ((), ())), preferred_element_type=f32)
    dotRT = lambda w, x: jax.lax.dot_general(
        w, x, (((0,), (1,)), ((), ())), preferred_element_type=f32)
    raw = flat_ref[0]
    raw = raw.reshape(_G, _N, 5).swapaxes(0, 1).reshape(_L, 5)

    h1 = _lrelu(dotRT(We1_ref[...], raw) + be1_ref[...])
    h = _lrelu(dotT(We2_ref[...], h1) + be2_ref[...])

    raw7 = jnp.concatenate([raw, raw[:, :2]], axis=1)
    tpart = (dotT(Wh1h_ref[...], h) - dotRT(Wh1t2_ref[...], raw[:, :2])
             + bh1_ref[...])
    kt = dotRT(Wkt_ref[...], raw)
    vt = dotRT(Wvt_ref[...], raw) + bv_ref[...]
    qs1 = dotT(Ws1q_ref[...], dotT(Wq_ref[...], h)) + bs1_ref[...]

    spart = dotRT(Wh1s_ref[...], raw7)
    ks = dotRT(Wks_ref[...], raw)
    vs = dotRT(Wvs_ref[...], raw)

    whard = whard_ref[...]
    ws2 = ws2_ref[...]
    lane_t = jax.lax.broadcasted_iota(jnp.int32, (1, _L), 1) // _G

    hard_rows, score_rows, v4s = [], [], []
    for s in range(_N):
        st = slice(s * _G, (s + 1) * _G)
        hh = jnp.maximum(tpart + pltpu.repeat(spart[:, st], _N, 1), 0.0)
        hard_rows.append(jnp.sum(hh * whard, axis=0, keepdims=True))
        k4 = _lrelu(kt + pltpu.repeat(ks[:, st], _N, 1))
        spre = jnp.maximum(dotT(Ws1k_ref[...], k4) + qs1, 0.0)
        sc = jnp.sum(spre * ws2, axis=0, keepdims=True) + bs2_ref[0, 0]
        score_rows.append(jnp.where(lane_t == s, -1e30, sc))
        v4s.append(_lrelu(vt + pltpu.repeat(vs[:, st], _N, 1)))

    scores = jnp.concatenate(score_rows, axis=0)
    m = jnp.max(scores, axis=0, keepdims=True)
    ex = jnp.exp(scores - m)
    hard_logit = jnp.concatenate(hard_rows, axis=0) + bhard_ref[0, 0]
    w = (ex / jnp.sum(ex, axis=0, keepdims=True)) * jax.nn.sigmoid(hard_logit)

    agg = w[0:1] * v4s[0]
    for s in range(1, _N):
        agg = agg + w[s:s + 1] * v4s[s]

    dec = _lrelu(dotT(Wd1h_ref[...], h) + dotT(Wd1o_ref[...], agg)
                 + bd1_ref[...])
    dec = _lrelu(dotT(Wd2_ref[...], dec) + bd2_ref[...])
    out_ref[0] = dec.T.reshape(_N, _G, 2 * _D).swapaxes(0, 1)


def kernel(embedding, W_e1, b_e1, W_e2, b_e2, W_h1, b_h1, W_h2, b_h2,
           W_he, b_he, W_q, W_k, W_v, b_v, W_s1, b_s1, W_s2, b_s2,
           W_d1, b_d1, W_d2, b_d2):
    flat_t = embedding.reshape(_GRID, _L, 5)

    whe_diff = W_he[:, 1] - W_he[:, 0]
    col1 = lambda b: b.reshape(-1, 1)
    weights = (
        W_e1, col1(b_e1), W_e2, col1(b_e2),
        W_h1[:_D], W_h1[_D + 5:],
        W_h1[_D:], col1(b_h1),
        (W_h2 @ whe_diff).reshape(_D, 1),
        (b_h2 @ whe_diff + b_he[1] - b_he[0]).reshape(1, 1),
        W_q,
        W_k[:5], W_k[5:],
        W_v[:5], W_v[5:], col1(b_v),
        W_s1[:_D], W_s1[_D:], col1(b_s1),
        W_s2.reshape(_D, 1), b_s2.reshape(1, 1),
        W_d1[:_D], W_d1[_D:], col1(b_d1),
        W_d2, col1(b_d2),
    )
    in_specs = [pl.BlockSpec((1, _L, 5), lambda i: (i, 0, 0))] + [
        pl.BlockSpec(w.shape, lambda i: (0, 0)) for w in weights
    ]
    out = pl.pallas_call(
        _body,
        grid=(_GRID,),
        in_specs=in_specs,
        out_specs=pl.BlockSpec((1, _G, _N, 2 * _D), lambda i: (i, 0, 0, 0)),
        out_shape=jax.ShapeDtypeStruct((_GRID, _G, _N, 2 * _D), jnp.float32),
        compiler_params=pltpu.CompilerParams(
            dimension_semantics=("parallel",)),
    )(flat_t, *weights)
    return out.reshape(_B * _N, 2 * _D)
